# Initial kernel scaffold; baseline (speedup 1.0000x reference)
import jax
import jax.numpy as jnp
from jax.experimental import pallas as pl


def kernel(x, w1, b1, w2, b2, se1, so1, s2, fc1w, fc1b, fc2w, fc2b):
    raise NotImplementedError("write your pallas kernel here")


# speedup vs baseline: 21.8442x; 21.8442x over previous
import jax
import jax.numpy as jnp
from jax.experimental import pallas as pl
from jax.experimental.pallas import tpu as pltpu

_CDT = jnp.bfloat16


def _net_kernel(xu_ref, w1_ref, b1_ref, w2_ref, b2_ref, fs_ref, fb1_ref,
                fw2_ref, fb2_ref, o_ref):
    G = o_ref.shape[0]
    f32 = jnp.float32

    c1 = jnp.dot(xu_ref[...], w1_ref[...], preferred_element_type=f32)
    m = jnp.maximum(jnp.maximum(c1[:, 0:256], c1[:, 256:512]),
                    jnp.maximum(c1[:, 512:768], c1[:, 768:1024]))
    y1 = jnp.maximum(m + b1_ref[...], 0.0)
    y1b = y1.astype(_CDT).reshape(G, 16, 256)

    acc = jnp.zeros((G * 8, 512), f32)
    for i in range(5):
        xi = y1b[:, i:i + 8, :].reshape(G * 8, 256)
        acc = acc + jnp.dot(xi, w2_ref[i], preferred_element_type=f32)
    p3 = acc.reshape(G, 8, 512)
    pm = jnp.maximum(p3[:, :, 0:256], p3[:, :, 256:512])

    h = jnp.zeros((G, 512), f32)
    for r in range(4):
        e = jnp.maximum(pm[:, 2 * r, :], pm[:, 2 * r + 1, :])
        y2 = jnp.maximum(e + b2_ref[...], 0.0)
        h = h + jnp.dot(y2.astype(_CDT), fs_ref[r],
                        preferred_element_type=f32)

    hr = jnp.maximum(h + fb1_ref[...], 0.0).astype(_CDT)
    z = jnp.dot(hr, fw2_ref[...], preferred_element_type=f32) + fb2_ref[...]
    mz = jnp.max(z, axis=-1, keepdims=True)
    ez = jnp.exp(z - mz)
    lse = jnp.log(jnp.sum(ez, axis=-1, keepdims=True)) + mz
    o_ref[...] = z - lse


def kernel(x, w1, b1, w2, b2, se1, so1, s2, fc1w, fc1b, fc2w, fc2b):
    del se1, so1, s2
    B = x.shape[0]
    G = 64
    while B % G:
        G //= 2
    f32 = jnp.float32

    xr = x.reshape(B, 28, 28)
    ridx = 2 * jnp.arange(12)[:, None] + jnp.arange(6)[None, :]
    xu = xr[:, ridx, :]
    xu = jnp.pad(xu, ((0, 0), (0, 4), (0, 0), (0, 0)))
    xu = xu.reshape(B * 16, 168).astype(_CDT)

    w1c = w1[:, :20].reshape(5, 5, 20)
    groups = []
    ow = jnp.arange(12)
    for rp in range(2):
        for wp in range(2):
            a = jnp.zeros((6, 28, 12, 20), f32)
            for i in range(5):
                for j in range(5):
                    a = a.at[i + rp, wp + j + 2 * ow, ow, :].set(
                        jnp.broadcast_to(w1c[i, j], (12, 20)))
            groups.append(jnp.pad(a.reshape(168, 240), ((0, 0), (0, 16))))
    w1all = jnp.concatenate(groups, axis=1).astype(_CDT)
    b1r = jnp.pad(jnp.tile(b1[0, :20], 12), (0, 16)).reshape(1, 256)

    w2c = w2[:, :20, :50].reshape(5, 5, 20, 50)
    ow2 = jnp.arange(4)
    taps = []
    for i in range(5):
        full = jnp.zeros((256, 512), f32)
        for wp in range(2):
            a = jnp.zeros((12, 20, 4, 50), f32)
            for j in range(5):
                a = a.at[wp + j + 2 * ow2, :, ow2, :].set(
                    jnp.broadcast_to(w2c[i, j], (4, 20, 50)))
            full = full.at[:240, wp * 256:wp * 256 + 200].set(
                a.reshape(240, 200))
        taps.append(full)
    w2s = jnp.stack(taps).astype(_CDT)
    b2r = jnp.pad(jnp.tile(b2[0, :50], 4), (0, 56)).reshape(1, 256)

    f3 = fc1w.reshape(16, 128, 512)[:, :50, :].reshape(4, 200, 512)
    fs = jnp.pad(f3, ((0, 0), (0, 56), (0, 0))).astype(_CDT)

    out = pl.pallas_call(
        _net_kernel,
        grid=(B // G,),
        out_shape=jax.ShapeDtypeStruct((B, 128), f32),
        in_specs=[
            pl.BlockSpec((G * 16, 168), lambda b: (b, 0)),
            pl.BlockSpec((168, 1024), lambda b: (0, 0)),
            pl.BlockSpec((1, 256), lambda b: (0, 0)),
            pl.BlockSpec((5, 256, 512), lambda b: (0, 0, 0)),
            pl.BlockSpec((1, 256), lambda b: (0, 0)),
            pl.BlockSpec((4, 256, 512), lambda b: (0, 0, 0)),
            pl.BlockSpec((1, 512), lambda b: (0, 0)),
            pl.BlockSpec((512, 128), lambda b: (0, 0)),
            pl.BlockSpec((1, 128), lambda b: (0, 0)),
        ],
        out_specs=pl.BlockSpec((G, 128), lambda b: (b, 0)),
        compiler_params=pltpu.CompilerParams(
            dimension_semantics=("parallel",)),
    )(xu, w1all, b1r, w2s, b2r, fs, fc1b, fc2w.astype(_CDT), fc2b)
    return out[:, :10]
